# baseline (speedup 1.0000x reference)
import functools

import jax
import jax.numpy as jnp
from jax import lax
from jax.experimental import pallas as pl
from jax.experimental.pallas import tpu as pltpu
from jax.experimental.pallas import tpu_sc as plsc

B, T, H = 4096, 200, 128

_info = plsc.get_sparse_core_info()
_NC, _NS, _L = _info.num_cores, _info.num_subcores, _info.num_lanes
_NW = _NC * _NS
_BPW = B // _NW


def _last_pool_kernel(flat_hbm, len_hbm, out_hbm, len_v, idx_v, rows_v, sem):
    wid = lax.axis_index("s") * _NC + lax.axis_index("c")
    base = wid * _BPW

    pltpu.sync_copy(len_hbm.at[pl.ds(base, _BPW)], len_v)

    lane = lax.iota(jnp.int32, _L)
    base_row = (base + lane) * T

    def body(i, row0):
        l = len_v[pl.ds(i * _L, _L)]
        t = jnp.where(l == 0, T - 1, l - 1)
        idx_v[pl.ds(i * _L, _L)] = row0 + t
        return row0 + _L * T

    lax.fori_loop(0, _BPW // _L, body, base_row, unroll=False)

    pltpu.async_copy(flat_hbm.at[idx_v], rows_v, sem).wait()
    pltpu.sync_copy(rows_v, out_hbm.at[pl.ds(base, _BPW)])


@functools.partial(
    pl.kernel,
    mesh=plsc.VectorSubcoreMesh(core_axis_name="c", subcore_axis_name="s"),
    out_type=jax.ShapeDtypeStruct((B, H), jnp.float32),
    scratch_types=[
        pltpu.VMEM((_BPW,), jnp.int32),
        pltpu.VMEM((_BPW,), jnp.int32),
        pltpu.VMEM((_BPW, H), jnp.float32),
        pltpu.SemaphoreType.DMA,
    ],
)
def _last_pool(flat_hbm, len_hbm, out_hbm, len_v, idx_v, rows_v, sem):
    _last_pool_kernel(flat_hbm, len_hbm, out_hbm, len_v, idx_v, rows_v, sem)


def kernel(inputs, length):
    flat = inputs.reshape(B * T, H)
    return _last_pool(flat, length.astype(jnp.int32))


# speedup vs baseline: 1.0054x; 1.0054x over previous
import functools

import jax
import jax.numpy as jnp
from jax import lax
from jax.experimental import pallas as pl
from jax.experimental.pallas import tpu as pltpu
from jax.experimental.pallas import tpu_sc as plsc

B, T, H = 4096, 200, 128

_info = plsc.get_sparse_core_info()
_NC, _NS, _L = _info.num_cores, _info.num_subcores, _info.num_lanes
_NW = _NC * _NS
_BPW = B // _NW


def _last_pool_kernel(flat_hbm, len_hbm, out_hbm, len_v, idx_v, rows_v, sem,
                      sem2, osem):
    wid = lax.axis_index("s") * _NC + lax.axis_index("c")
    base = wid * _BPW

    pltpu.sync_copy(len_hbm.at[pl.ds(base, _BPW)], len_v)

    lane = lax.iota(jnp.int32, _L)
    base_row = (base + lane) * T

    def body(i, row0):
        l = len_v[pl.ds(i * _L, _L)]
        t = jnp.where(l == 0, T - 1, l - 1)
        idx_v[pl.ds(i * _L, _L)] = row0 + t
        return row0 + _L * T

    lax.fori_loop(0, _BPW // _L, body, base_row, unroll=False)

    c0 = _BPW // 4
    g0 = pltpu.async_copy(
        flat_hbm.at[idx_v.at[pl.ds(0, c0)]], rows_v.at[pl.ds(0, c0)], sem)
    g1 = pltpu.async_copy(
        flat_hbm.at[idx_v.at[pl.ds(c0, _BPW - c0)]],
        rows_v.at[pl.ds(c0, _BPW - c0)], sem2)
    g0.wait()
    o0 = pltpu.async_copy(
        rows_v.at[pl.ds(0, c0)], out_hbm.at[pl.ds(base, c0)], osem)
    g1.wait()
    o1 = pltpu.async_copy(
        rows_v.at[pl.ds(c0, _BPW - c0)],
        out_hbm.at[pl.ds(base + c0, _BPW - c0)], osem)
    o0.wait()
    o1.wait()


@functools.partial(
    pl.kernel,
    mesh=plsc.VectorSubcoreMesh(core_axis_name="c", subcore_axis_name="s"),
    out_type=jax.ShapeDtypeStruct((B, H), jnp.float32),
    scratch_types=[
        pltpu.VMEM((_BPW,), jnp.int32),
        pltpu.VMEM((_BPW,), jnp.int32),
        pltpu.VMEM((_BPW, H), jnp.float32),
        pltpu.SemaphoreType.DMA,
        pltpu.SemaphoreType.DMA,
        pltpu.SemaphoreType.DMA,
    ],
)
def _last_pool(flat_hbm, len_hbm, out_hbm, len_v, idx_v, rows_v, sem, sem2,
               osem):
    _last_pool_kernel(flat_hbm, len_hbm, out_hbm, len_v, idx_v, rows_v, sem,
                      sem2, osem)


def kernel(inputs, length):
    flat = inputs.reshape(B * T, H)
    return _last_pool(flat, length.astype(jnp.int32))
